# baseline (speedup 1.0000x reference)
import jax
import jax.numpy as jnp
from jax.experimental import pallas as pl
from jax.experimental.pallas import tpu as pltpu

N = 4096
D = 512
BJ = 256
NJ = N // BJ
NP = 8
BI = N // NP
NEG_SLOPE = 0.2
LOG2E = 1.4426950408889634
NEG_BIG = -1e30


def _lrelu(x):
    return jnp.maximum(x, NEG_SLOPE * x)


def _body(x_ref, w_ref, asrc_ref, adst_ref, a1_ref, a2_ref, bias_ref, out_ref,
          xp_ref, as_ref, ad_ref, gmax_ref):
    t = pl.program_id(0)

    @pl.when(t < NP)
    def _proj():
        i = t
        xp = jax.lax.dot_general(
            x_ref[...].astype(jnp.bfloat16), w_ref[...].astype(jnp.bfloat16),
            (((1,), (0,)), ((), ())), preferred_element_type=jnp.float32)
        xp_ref[pl.ds(i * BI, BI), :] = xp
        as_blk = jax.lax.dot_general(
            xp, asrc_ref[...] * LOG2E, (((1,), (0,)), ((), ())),
            preferred_element_type=jnp.float32)
        as_ref[pl.ds(i * BI, BI), :] = as_blk
        ad_ref[0:1, pl.ds(i * BI, BI)] = jax.lax.dot_general(
            adst_ref[...] * LOG2E, xp, (((0,), (1,)), ((), ())),
            preferred_element_type=jnp.float32)
        prev = jnp.where(i == 0, jnp.full((1, 1), NEG_BIG, jnp.float32),
                         gmax_ref[...])
        gmax_ref[...] = jnp.maximum(prev, jnp.max(as_blk))

    @pl.when(t >= NP)
    def _agg():
        j = t - NP
        ad_row = ad_ref[0:1, pl.ds(j * BJ, BJ)]
        m2 = _lrelu(gmax_ref[...] + ad_row)
        nh = N // 2

        def _p(a_h, row0):
            z = as_ref[pl.ds(row0, nh), :] + ad_row
            e2 = _lrelu(z)
            return a_h[...] * jnp.exp2(e2 - m2)

        p1 = _p(a1_ref, 0)
        p2 = _p(a2_ref, nh)
        ones = jnp.ones((nh, 1), jnp.float32)
        out = (jax.lax.dot_general(
                   p1, xp_ref[pl.ds(0, nh), :], (((0,), (0,)), ((), ())),
                   preferred_element_type=jnp.float32) +
               jax.lax.dot_general(
                   p2, xp_ref[pl.ds(nh, nh), :], (((0,), (0,)), ((), ())),
                   preferred_element_type=jnp.float32))
        s = (jax.lax.dot_general(
                 p1, ones, (((0,), (0,)), ((), ())),
                 preferred_element_type=jnp.float32) +
             jax.lax.dot_general(
                 p2, ones, (((0,), (0,)), ((), ())),
                 preferred_element_type=jnp.float32))
        s_safe = jnp.where(s > 0.0, s, 1.0)
        out_ref[...] = jnp.maximum(out / s_safe + bias_ref[...], 0.0)


@jax.jit
def kernel(A, X, W, a_src, a_dst, bias):
    d_in = X.shape[1]
    out = pl.pallas_call(
        _body,
        grid=(NP + NJ,),
        in_specs=[
            pl.BlockSpec((BI, d_in), lambda t: (jnp.minimum(t, NP - 1), 0)),
            pl.BlockSpec((d_in, D), lambda t: (0, 0)),
            pl.BlockSpec((D, 1), lambda t: (0, 0)),
            pl.BlockSpec((D, 1), lambda t: (0, 0)),
            pl.BlockSpec((N // 2, BJ), lambda t: (0, jnp.maximum(t - NP, 0))),
            pl.BlockSpec((N // 2, BJ), lambda t: (1, jnp.maximum(t - NP, 0))),
            pl.BlockSpec((1, D), lambda t: (0, 0)),
        ],
        out_specs=pl.BlockSpec((BJ, D), lambda t: (jnp.maximum(t - NP, 0), 0)),
        out_shape=jax.ShapeDtypeStruct((N, D), jnp.float32),
        scratch_shapes=[
            pltpu.VMEM((N, D), jnp.float32),
            pltpu.VMEM((N, 1), jnp.float32),
            pltpu.VMEM((1, N), jnp.float32),
            pltpu.VMEM((1, 1), jnp.float32),
        ],
        compiler_params=pltpu.CompilerParams(
            dimension_semantics=("arbitrary",)),
    )(X, W, a_src.reshape(D, 1), a_dst.reshape(D, 1), A, A,
      bias.reshape(1, D))

    return out


# speedup vs baseline: 1.0919x; 1.0919x over previous
import jax
import jax.numpy as jnp
from jax.experimental import pallas as pl
from jax.experimental.pallas import tpu as pltpu

N = 4096
D = 512
BJ = 512
NJ = N // BJ
NP = 8
BI = N // NP
NEG_SLOPE = 0.2
LOG2E = 1.4426950408889634
NEG_BIG = -1e30


def _lrelu(x):
    return jnp.maximum(x, NEG_SLOPE * x)


def _body(x_ref, w_ref, asrc_ref, adst_ref, a1_ref, a2_ref, a3_ref, a4_ref,
          bias_ref, out_ref,
          xp_ref, as_ref, ad_ref, gmax_ref):
    t = pl.program_id(0)

    @pl.when(t < NP)
    def _proj():
        i = t
        xp = jax.lax.dot_general(
            x_ref[...].astype(jnp.bfloat16), w_ref[...].astype(jnp.bfloat16),
            (((1,), (0,)), ((), ())), preferred_element_type=jnp.float32)
        xp_ref[pl.ds(i * BI, BI), :] = xp
        as_blk = jax.lax.dot_general(
            xp, asrc_ref[...] * LOG2E, (((1,), (0,)), ((), ())),
            preferred_element_type=jnp.float32)
        as_ref[pl.ds(i * BI, BI), :] = as_blk
        ad_ref[0:1, pl.ds(i * BI, BI)] = jax.lax.dot_general(
            adst_ref[...] * LOG2E, xp, (((0,), (1,)), ((), ())),
            preferred_element_type=jnp.float32)
        prev = jnp.where(i == 0, jnp.full((1, 1), NEG_BIG, jnp.float32),
                         gmax_ref[...])
        gmax_ref[...] = jnp.maximum(prev, jnp.max(as_blk))

    @pl.when(t >= NP)
    def _agg():
        j = t - NP
        ad_row = ad_ref[0:1, pl.ds(j * BJ, BJ)]
        m2 = _lrelu(gmax_ref[...] + ad_row)
        nh = N // 4
        ones = jnp.ones((nh, 1), jnp.float32)

        def _contrib(a_h, k):
            z = as_ref[pl.ds(k * nh, nh), :] + ad_row
            e2 = _lrelu(z)
            p = a_h[...] * jnp.exp2(e2 - m2)
            o = jax.lax.dot_general(
                p, xp_ref[pl.ds(k * nh, nh), :], (((0,), (0,)), ((), ())),
                preferred_element_type=jnp.float32)
            sc = jax.lax.dot_general(
                p, ones, (((0,), (0,)), ((), ())),
                preferred_element_type=jnp.float32)
            return o, sc

        o1, s1 = _contrib(a1_ref, 0)
        o2, s2 = _contrib(a2_ref, 1)
        o3, s3 = _contrib(a3_ref, 2)
        o4, s4 = _contrib(a4_ref, 3)
        out = (o1 + o2) + (o3 + o4)
        s = (s1 + s2) + (s3 + s4)
        s_safe = jnp.where(s > 0.0, s, 1.0)
        out_ref[...] = jnp.maximum(out / s_safe + bias_ref[...], 0.0)


@jax.jit
def kernel(A, X, W, a_src, a_dst, bias):
    d_in = X.shape[1]
    out = pl.pallas_call(
        _body,
        grid=(NP + NJ,),
        in_specs=[
            pl.BlockSpec((BI, d_in), lambda t: (jnp.minimum(t, NP - 1), 0)),
            pl.BlockSpec((d_in, D), lambda t: (0, 0)),
            pl.BlockSpec((D, 1), lambda t: (0, 0)),
            pl.BlockSpec((D, 1), lambda t: (0, 0)),
            pl.BlockSpec((N // 4, BJ), lambda t: (0, jnp.maximum(t - NP, 0))),
            pl.BlockSpec((N // 4, BJ), lambda t: (1, jnp.maximum(t - NP, 0))),
            pl.BlockSpec((N // 4, BJ), lambda t: (2, jnp.maximum(t - NP, 0))),
            pl.BlockSpec((N // 4, BJ), lambda t: (3, jnp.maximum(t - NP, 0))),
            pl.BlockSpec((1, D), lambda t: (0, 0)),
        ],
        out_specs=pl.BlockSpec((BJ, D), lambda t: (jnp.maximum(t - NP, 0), 0)),
        out_shape=jax.ShapeDtypeStruct((N, D), jnp.float32),
        scratch_shapes=[
            pltpu.VMEM((N, D), jnp.float32),
            pltpu.VMEM((N, 1), jnp.float32),
            pltpu.VMEM((1, N), jnp.float32),
            pltpu.VMEM((1, 1), jnp.float32),
        ],
        compiler_params=pltpu.CompilerParams(
            dimension_semantics=("arbitrary",)),
    )(X, W, a_src.reshape(D, 1), a_dst.reshape(D, 1), A, A, A, A,
      bias.reshape(1, D))

    return out
